# baseline (speedup 1.0000x reference)
import functools

import jax
import jax.numpy as jnp
from jax import lax
from jax.experimental import pallas as pl
from jax.experimental.pallas import tpu as pltpu
from jax.experimental.pallas import tpu_sc as plsc

N = 10000
NPAD = 10240
E = 320000
E2 = E // 2
H = 128
HH = H // 2
D_EDGE = 16
MP = 5

TRE = 512
TRH = 400
TRN = 512

NW = 32
CH = 40
PW = E2 // NW
NCHG = PW // CH
GSLOT = 5
GLOOK = 2
PWS = E2 // 16
CHS = 40
NCHS = PWS // CHS
NSLOT = 10
LOOK = NSLOT // 2
RPT = NPAD // 16


def _ln(xx, g, beta):
    m = jnp.mean(xx, axis=-1, keepdims=True)
    v = jnp.mean((xx - m) ** 2, axis=-1, keepdims=True)
    return (xx - m) * lax.rsqrt(v + 1e-5) * g + beta


def _dot(a, b):
    return jnp.dot(a, b, preferred_element_type=jnp.float32)


def _pack(xf32):
    lo = lax.bitcast_convert_type(xf32[:, :HH].astype(jnp.bfloat16),
                                  jnp.uint16).astype(jnp.uint32)
    hi = lax.bitcast_convert_type(xf32[:, HH:].astype(jnp.bfloat16),
                                  jnp.uint16).astype(jnp.uint32)
    return lo | (hi << 16)


def _unpack(w):
    lo = lax.bitcast_convert_type((w & jnp.uint32(0xFFFF)).astype(jnp.uint16),
                                  jnp.bfloat16)
    hi = lax.bitcast_convert_type((w >> 16).astype(jnp.uint16), jnp.bfloat16)
    return jnp.concatenate([lo, hi], axis=-1).astype(jnp.float32)


def _bn_stats_body(x_ref, o_ref):
    i = pl.program_id(0)
    xb = x_ref[...]
    part = jnp.concatenate(
        [jnp.sum(xb, axis=0, keepdims=True),
         jnp.sum(xb * xb, axis=0, keepdims=True)], axis=0)

    @pl.when(i == 0)
    def _():
        o_ref[...] = part

    @pl.when(i > 0)
    def _():
        o_ref[...] += part


_bn_stats = pl.pallas_call(
    _bn_stats_body,
    grid=(E // TRE,),
    in_specs=[pl.BlockSpec((TRE, D_EDGE), lambda i: (i, 0))],
    out_specs=pl.BlockSpec((2, D_EDGE), lambda i: (0, 0)),
    out_shape=jax.ShapeDtypeStruct((2, D_EDGE), jnp.float32),
)


def _w_specs(shapes):
    return [pl.BlockSpec(s, lambda i: tuple(0 for _ in s)) for s in shapes]


def _enc_edge_body(x_ref, w1, b1, w2, b2, w3, b3, g, beta, o_ref):
    l1 = jax.nn.relu(_dot(x_ref[...], w1[...]) + b1[...])
    l2 = jax.nn.relu(_dot(l1, w2[...]) + b2[...])
    l3 = _dot(l2, w3[...]) + b3[...]
    o_ref[...] = _ln(l3, g[...], beta[...])


_enc_edge = pl.pallas_call(
    _enc_edge_body,
    grid=(E2 // TRH,),
    in_specs=[pl.BlockSpec((TRH, D_EDGE), lambda i: (i, 0))]
    + _w_specs([(D_EDGE, H), (1, H), (H, H), (1, H), (H, H), (1, H), (1, H),
                (1, H)]),
    out_specs=pl.BlockSpec((TRH, H), lambda i: (i, 0)),
    out_shape=jax.ShapeDtypeStruct((E2, H), jnp.float32),
)


def _enc_node_body(x_ref, w1, b1, w2, b2, w3, b3, g, beta, wa, wb,
                   h_ref, hab_ref):
    l1 = jax.nn.relu(_dot(x_ref[...], w1[...]) + b1[...])
    l2 = jax.nn.relu(_dot(l1, w2[...]) + b2[...])
    l3 = _dot(l2, w3[...]) + b3[...]
    h = _ln(l3, g[...], beta[...])
    h_ref[...] = h
    hab_ref[...] = jnp.concatenate(
        [_pack(_dot(h, wa[...])), _pack(_dot(h, wb[...]))], axis=-1)


_enc_node = pl.pallas_call(
    _enc_node_body,
    grid=(NPAD // TRN,),
    in_specs=[pl.BlockSpec((TRN, H), lambda i: (i, 0))]
    + _w_specs([(H, H), (1, H), (H, H), (1, H), (H, H), (1, H), (1, H), (1, H),
                (H, H), (H, H)]),
    out_specs=[pl.BlockSpec((TRN, H), lambda i: (i, 0)),
               pl.BlockSpec((TRN, H), lambda i: (i, 0))],
    out_shape=[jax.ShapeDtypeStruct((NPAD, H), jnp.float32),
               jax.ShapeDtypeStruct((NPAD, H), jnp.uint32)],
)


def _edge_mlp_body(ga_ref, gb_ref, e_ref, w1c, b1, w2, b2, w3, b3, g, beta, wm,
                   e_out, m_out):
    e_in = e_ref[...]
    ga = _unpack(ga_ref[...][:, :HH])
    gb = _unpack(gb_ref[...][:, HH:])
    l1 = jax.nn.relu(ga + gb + _dot(e_in, w1c[...]) + b1[...])
    l2 = jax.nn.relu(_dot(l1, w2[...]) + b2[...])
    l3 = _dot(l2, w3[...]) + b3[...]
    e_new = _ln(l3, g[...], beta[...]) + e_in
    e_out[...] = e_new
    mm = _dot(e_new, wm[...])
    m_out[0] = mm[:, :HH]
    m_out[1] = mm[:, HH:]


_edge_mlp = pl.pallas_call(
    _edge_mlp_body,
    grid=(E2 // TRH,),
    in_specs=[pl.BlockSpec((TRH, H), lambda i: (i, 0))] * 3
    + _w_specs([(H, H), (1, H), (H, H), (1, H), (H, H), (1, H), (1, H), (1, H),
                (H, H)]),
    out_specs=[pl.BlockSpec((TRH, H), lambda i: (i, 0)),
               pl.BlockSpec((2, TRH, HH), lambda i: (0, i, 0))],
    out_shape=[jax.ShapeDtypeStruct((E2, H), jnp.float32),
               jax.ShapeDtypeStruct((2, E2, HH), jnp.float32)],
)


def _agg_cat(a0h1, a1h1, a0h2, a1h2):
    return jnp.concatenate([a0h1[0] + a0h2[0], a1h1[0] + a1h2[0]], axis=-1)


_AGG_SPECS = [pl.BlockSpec((1, TRN, HH), lambda i: (0, i, 0)),
              pl.BlockSpec((1, TRN, HH), lambda i: (1, i, 0)),
              pl.BlockSpec((1, TRN, HH), lambda i: (0, i, 0)),
              pl.BlockSpec((1, TRN, HH), lambda i: (1, i, 0))]


def _node_mlp_body(h_ref, a0_ref, a1_ref, a2_ref, a3_ref,
                   w1, b1, w2, b2, w3, b3, g, beta, wa, wb, h_out, hab_out):
    h_in = h_ref[...]
    agg = _agg_cat(a0_ref, a1_ref, a2_ref, a3_ref)
    l1 = jax.nn.relu(_dot(h_in, w1[...]) + agg + b1[...])
    l2 = jax.nn.relu(_dot(l1, w2[...]) + b2[...])
    l3 = _dot(l2, w3[...]) + b3[...]
    h_new = _ln(l3, g[...], beta[...]) + h_in
    h_out[...] = h_new
    hab_out[...] = jnp.concatenate(
        [_pack(_dot(h_new, wa[...])), _pack(_dot(h_new, wb[...]))], axis=-1)


_node_mlp = pl.pallas_call(
    _node_mlp_body,
    grid=(NPAD // TRN,),
    in_specs=[pl.BlockSpec((TRN, H), lambda i: (i, 0))] + _AGG_SPECS
    + _w_specs([(H, H), (1, H), (H, H), (1, H), (H, H), (1, H), (1, H), (1, H),
                (H, H), (H, H)]),
    out_specs=[pl.BlockSpec((TRN, H), lambda i: (i, 0)),
               pl.BlockSpec((TRN, H), lambda i: (i, 0))],
    out_shape=[jax.ShapeDtypeStruct((NPAD, H), jnp.float32),
               jax.ShapeDtypeStruct((NPAD, H), jnp.uint32)],
)


def _node_last_body(h_ref, a0_ref, a1_ref, a2_ref, a3_ref,
                    w1, b1, w2, b2, w3, b3, g, beta, h_out):
    h_in = h_ref[...]
    agg = _agg_cat(a0_ref, a1_ref, a2_ref, a3_ref)
    l1 = jax.nn.relu(_dot(h_in, w1[...]) + agg + b1[...])
    l2 = jax.nn.relu(_dot(l1, w2[...]) + b2[...])
    l3 = _dot(l2, w3[...]) + b3[...]
    h_out[...] = _ln(l3, g[...], beta[...]) + h_in


_node_last = pl.pallas_call(
    _node_last_body,
    grid=(NPAD // TRN,),
    in_specs=[pl.BlockSpec((TRN, H), lambda i: (i, 0))] + _AGG_SPECS
    + _w_specs([(H, H), (1, H), (H, H), (1, H), (H, H), (1, H), (1, H),
                (1, H)]),
    out_specs=pl.BlockSpec((TRN, H), lambda i: (i, 0)),
    out_shape=jax.ShapeDtypeStruct((NPAD, H), jnp.float32),
)


def _dec_body(h_ref, x_ref, w1, b1, w2, b2, w3, b3, o_ref):
    l1 = jax.nn.relu(_dot(h_ref[...], w1[...]) + b1[...])
    l2 = jax.nn.relu(_dot(l1, w2[...]) + b2[...])
    l3 = _dot(l2, w3[...]) + b3[...]
    o_ref[...] = l3 * 0.005 + x_ref[:, :3]


_dec = pl.pallas_call(
    _dec_body,
    grid=(NPAD // TRN,),
    in_specs=[pl.BlockSpec((TRN, H), lambda i: (i, 0)),
              pl.BlockSpec((TRN, H), lambda i: (i, 0))]
    + _w_specs([(H, H), (1, H), (H, H), (1, H), (H, 3), (1, 3)]),
    out_specs=pl.BlockSpec((TRN, 3), lambda i: (i, 0)),
    out_shape=jax.ShapeDtypeStruct((NPAD, 3), jnp.float32),
)


_sc_mesh = plsc.VectorSubcoreMesh(core_axis_name="c", subcore_axis_name="s")


@functools.partial(
    pl.kernel,
    out_type=[jax.ShapeDtypeStruct((E2, H), jnp.uint32),
              jax.ShapeDtypeStruct((E2, H), jnp.uint32)],
    mesh=_sc_mesh,
    scratch_types=[pltpu.VMEM((GSLOT, CH), jnp.int32),
                   pltpu.VMEM((GSLOT, CH, H), jnp.uint32),
                   pltpu.SemaphoreType.DMA((GSLOT,)),
                   pltpu.SemaphoreType.DMA((GSLOT,)),
                   pltpu.SemaphoreType.DMA((GSLOT,))],
)
def _sc_gather(hab_hbm, idxg_hbm, ga_hbm, gb_hbm, idx, buf, isem, gsem, wsem):
    cid = lax.axis_index("c")
    sid = lax.axis_index("s")
    wid = sid * 2 + cid
    base = wid * PW

    for phase in range(2):
        cbase = phase * NCHG
        out = (ga_hbm, gb_hbm)[phase]

        def _wait_write(b, _out=out):
            pltpu.make_async_copy(buf.at[b], _out.at[pl.ds(base, CH)],
                                  wsem.at[b]).wait()

        for b in range(GSLOT):
            pltpu.async_copy(idxg_hbm.at[wid, cbase + b], idx.at[b],
                             isem.at[b])
        for b in range(GLOOK):
            pltpu.make_async_copy(idxg_hbm.at[wid, 0], idx.at[b],
                                  isem.at[b]).wait()
            pltpu.async_copy(hab_hbm.at[idx.at[b]], buf.at[b], gsem.at[b])

        @pl.loop(0, NCHG, step=GSLOT)
        def _round(step, _out=out, _cbase=cbase, _wait_write=_wait_write):
            for b in range(GSLOT):
                v = step + b
                ub = (b + GLOOK) % GSLOT
                pltpu.make_async_copy(hab_hbm.at[idx.at[b]], buf.at[b],
                                      gsem.at[b]).wait()
                pltpu.async_copy(buf.at[b],
                                 _out.at[pl.ds(base + v * CH, CH)],
                                 wsem.at[b])
                cw = v + GSLOT

                @pl.when(cw < NCHG)
                def _():
                    pltpu.async_copy(idxg_hbm.at[wid, _cbase + cw], idx.at[b],
                                     isem.at[b])

                u = v + GLOOK

                @pl.when(u < NCHG)
                def _():
                    @pl.when(u >= GSLOT)
                    def _():
                        _wait_write(ub)

                    pltpu.make_async_copy(idxg_hbm.at[wid, 0], idx.at[ub],
                                          isem.at[ub]).wait()
                    pltpu.async_copy(hab_hbm.at[idx.at[ub]], buf.at[ub],
                                     gsem.at[ub])

        for b in range(GSLOT):
            _wait_write(b)


@functools.partial(
    pl.kernel,
    out_type=jax.ShapeDtypeStruct((2, NPAD, HH), jnp.float32),
    mesh=_sc_mesh,
    scratch_types=[pltpu.VMEM_SHARED((NPAD, HH), jnp.float32),
                   pltpu.VMEM((NSLOT, CHS), jnp.int32),
                   pltpu.VMEM((NSLOT, CHS, HH), jnp.float32),
                   pltpu.SemaphoreType.DMA((NSLOT,)),
                   pltpu.SemaphoreType.DMA((NSLOT,)),
                   pltpu.SemaphoreType.DMA((NSLOT,))],
)
def _sc_scatter(ms_hbm, cols_hbm, zeros_hbm, agg_hbm, shared, idx, buf,
                isem, lsem, asem):
    cid = lax.axis_index("c")
    sid = lax.axis_index("s")
    base = sid * PWS
    pltpu.sync_copy(zeros_hbm.at[pl.ds(sid * RPT, RPT)],
                    shared.at[pl.ds(sid * RPT, RPT)])
    plsc.subcore_barrier()
    for b in range(LOOK):
        pltpu.async_copy(cols_hbm.at[sid, b], idx.at[b], isem.at[b])
        pltpu.async_copy(ms_hbm.at[cid, pl.ds(base + b * CHS, CHS)],
                         buf.at[b], lsem.at[b])

    @pl.loop(0, NCHS, step=NSLOT)
    def _round(step):
        for b in range(NSLOT):
            v = step + b
            ub = (b + LOOK) % NSLOT
            pltpu.make_async_copy(cols_hbm.at[sid, 0], idx.at[b],
                                  isem.at[b]).wait()
            pltpu.make_async_copy(ms_hbm.at[cid, pl.ds(base, CHS)], buf.at[b],
                                  lsem.at[b]).wait()
            pltpu.async_copy(buf.at[b], shared.at[idx.at[b]], asem.at[b],
                             add=True)
            u = v + LOOK

            @pl.when(u < NCHS)
            def _():
                @pl.when(u >= NSLOT)
                def _():
                    pltpu.make_async_copy(buf.at[ub], shared.at[idx.at[ub]],
                                          asem.at[ub]).wait()

                pltpu.async_copy(cols_hbm.at[sid, u], idx.at[ub],
                                 isem.at[ub])
                pltpu.async_copy(ms_hbm.at[cid, pl.ds(base + u * CHS, CHS)],
                                 buf.at[ub], lsem.at[ub])

    for b in range(NSLOT):
        pltpu.make_async_copy(buf.at[b], shared.at[idx.at[b]],
                              asem.at[b]).wait()
    plsc.subcore_barrier()
    pltpu.sync_copy(shared.at[pl.ds(sid * RPT, RPT)],
                    agg_hbm.at[cid, pl.ds(sid * RPT, RPT)])


def _r(v):
    return v.reshape(1, -1)


def kernel(x, edge_index, edge_attr, params):
    row2 = edge_index[0].reshape(2, E2)
    col2 = edge_index[1].reshape(2, E2)
    idxg = [jnp.concatenate([row2[hf].reshape(NW, NCHG, CH),
                             col2[hf].reshape(NW, NCHG, CH)], axis=1)
            for hf in range(2)]
    cols = [col2[hf].reshape(16, NCHS, CHS) for hf in range(2)]
    ea2 = edge_attr.reshape(2, E2, D_EDGE)
    xp = jnp.pad(x, ((0, NPAD - N), (0, 0)))
    zeros_n = jnp.zeros((NPAD, HH), jnp.float32)

    sums = _bn_stats(edge_attr)
    mu = sums[0] / E
    var = sums[1] / E - mu * mu
    s = params["bn"]["gamma"] * lax.rsqrt(var + 1e-5)
    ep = params["edge_enc"]
    w1p = ep["W"][0] * s[:, None]
    b1p = ep["b"][0] + (params["bn"]["beta"] - mu * s) @ ep["W"][0]
    e = [_enc_edge(ea2[hf], w1p, _r(b1p), ep["W"][1], _r(ep["b"][1]),
                   ep["W"][2], _r(ep["b"][2]), _r(ep["g"]), _r(ep["beta"]))
         for hf in range(2)]

    np_ = params["node_enc"]
    we0 = params["blocks"][0]["edge"]["W"][0]
    h, hab = _enc_node(xp, np_["W"][0], _r(np_["b"][0]),
                       np_["W"][1], _r(np_["b"][1]),
                       np_["W"][2], _r(np_["b"][2]),
                       _r(np_["g"]), _r(np_["beta"]),
                       we0[:H], we0[H:2 * H])

    for i in range(MP):
        blk = params["blocks"][i]
        pe, pn = blk["edge"], blk["node"]
        eargs = (pe["W"][0][2 * H:], _r(pe["b"][0]),
                 pe["W"][1], _r(pe["b"][1]), pe["W"][2], _r(pe["b"][2]),
                 _r(pe["g"]), _r(pe["beta"]), pn["W"][0][H:])
        aggs = []
        for hf in range(2):
            ga, gb = _sc_gather(hab, idxg[hf])
            e[hf], ms = _edge_mlp(ga, gb, e[hf], *eargs)
            aggs.append(_sc_scatter(ms, cols[hf], zeros_n))
        nargs = (h, aggs[0], aggs[0], aggs[1], aggs[1],
                 pn["W"][0][:H], _r(pn["b"][0]),
                 pn["W"][1], _r(pn["b"][1]), pn["W"][2], _r(pn["b"][2]),
                 _r(pn["g"]), _r(pn["beta"]))
        if i < MP - 1:
            wen = params["blocks"][i + 1]["edge"]["W"][0]
            h, hab = _node_mlp(*nargs, wen[:H], wen[H:2 * H])
        else:
            h = _node_last(*nargs)

    d = params["dec"]
    out = _dec(h, xp, d["W"][0], _r(d["b"][0]), d["W"][1], _r(d["b"][1]),
               d["W"][2], _r(d["b"][2]))
    return out[:N]


# speedup vs baseline: 1.0318x; 1.0318x over previous
import functools

import jax
import jax.numpy as jnp
from jax import lax
from jax.experimental import pallas as pl
from jax.experimental.pallas import tpu as pltpu
from jax.experimental.pallas import tpu_sc as plsc

N = 10000
NPAD = 10240
E = 320000
E2 = E
H = 128
HH = H // 2
D_EDGE = 16
MP = 5

TRE = 512
TRH = 512
TRN = 512

NW = 32
CH = 40
PW = E2 // NW
NCHG = PW // CH
GSLOT = 5
GLOOK = 2
PWS = E2 // 16
CHS = 40
NCHS = PWS // CHS
NSLOT = 10
LOOK = NSLOT // 2
RPT = NPAD // 16


def _ln(xx, g, beta):
    m = jnp.mean(xx, axis=-1, keepdims=True)
    v = jnp.mean((xx - m) ** 2, axis=-1, keepdims=True)
    return (xx - m) * lax.rsqrt(v + 1e-5) * g + beta


def _dot(a, b):
    return jnp.dot(a.astype(jnp.bfloat16), b.astype(jnp.bfloat16),
                   preferred_element_type=jnp.float32)


def _pack(xf32):
    lo = lax.bitcast_convert_type(xf32[:, :HH].astype(jnp.bfloat16),
                                  jnp.uint16).astype(jnp.uint32)
    hi = lax.bitcast_convert_type(xf32[:, HH:].astype(jnp.bfloat16),
                                  jnp.uint16).astype(jnp.uint32)
    return lo | (hi << 16)


def _unpack(w):
    lo = lax.bitcast_convert_type((w & jnp.uint32(0xFFFF)).astype(jnp.uint16),
                                  jnp.bfloat16)
    hi = lax.bitcast_convert_type((w >> 16).astype(jnp.uint16), jnp.bfloat16)
    return jnp.concatenate([lo, hi], axis=-1).astype(jnp.float32)


def _bn_stats_body(x_ref, o_ref):
    i = pl.program_id(0)
    xb = x_ref[...]
    part = jnp.concatenate(
        [jnp.sum(xb, axis=0, keepdims=True),
         jnp.sum(xb * xb, axis=0, keepdims=True)], axis=0)

    @pl.when(i == 0)
    def _():
        o_ref[...] = part

    @pl.when(i > 0)
    def _():
        o_ref[...] += part


_bn_stats = pl.pallas_call(
    _bn_stats_body,
    grid=(E // TRE,),
    in_specs=[pl.BlockSpec((TRE, D_EDGE), lambda i: (i, 0))],
    out_specs=pl.BlockSpec((2, D_EDGE), lambda i: (0, 0)),
    out_shape=jax.ShapeDtypeStruct((2, D_EDGE), jnp.float32),
)


def _w_specs(shapes):
    return [pl.BlockSpec(s, lambda i: tuple(0 for _ in s)) for s in shapes]


def _enc_edge_body(x_ref, w1, b1, w2, b2, w3, b3, g, beta, o_ref):
    l1 = jax.nn.relu(_dot(x_ref[...], w1[...]) + b1[...])
    l2 = jax.nn.relu(_dot(l1, w2[...]) + b2[...])
    l3 = _dot(l2, w3[...]) + b3[...]
    o_ref[...] = _ln(l3, g[...], beta[...])


_enc_edge = pl.pallas_call(
    _enc_edge_body,
    grid=(E2 // TRH,),
    in_specs=[pl.BlockSpec((TRH, D_EDGE), lambda i: (i, 0))]
    + _w_specs([(D_EDGE, H), (1, H), (H, H), (1, H), (H, H), (1, H), (1, H),
                (1, H)]),
    out_specs=pl.BlockSpec((TRH, H), lambda i: (i, 0)),
    out_shape=jax.ShapeDtypeStruct((E2, H), jnp.float32),
)


def _enc_node_body(x_ref, w1, b1, w2, b2, w3, b3, g, beta, wa, wb,
                   h_ref, hab_ref):
    l1 = jax.nn.relu(_dot(x_ref[...], w1[...]) + b1[...])
    l2 = jax.nn.relu(_dot(l1, w2[...]) + b2[...])
    l3 = _dot(l2, w3[...]) + b3[...]
    h = _ln(l3, g[...], beta[...])
    h_ref[...] = h
    hab_ref[...] = jnp.concatenate(
        [_pack(_dot(h, wa[...])), _pack(_dot(h, wb[...]))], axis=-1)


_enc_node = pl.pallas_call(
    _enc_node_body,
    grid=(NPAD // TRN,),
    in_specs=[pl.BlockSpec((TRN, H), lambda i: (i, 0))]
    + _w_specs([(H, H), (1, H), (H, H), (1, H), (H, H), (1, H), (1, H), (1, H),
                (H, H), (H, H)]),
    out_specs=[pl.BlockSpec((TRN, H), lambda i: (i, 0)),
               pl.BlockSpec((TRN, H), lambda i: (i, 0))],
    out_shape=[jax.ShapeDtypeStruct((NPAD, H), jnp.float32),
               jax.ShapeDtypeStruct((NPAD, H), jnp.uint32)],
)


def _edge_mlp_body(ga_ref, gb_ref, e_ref, w1c, b1, w2, b2, w3, b3, g, beta, wm,
                   e_out, m_out):
    e_in = e_ref[...]
    ga = _unpack(ga_ref[...][:, :HH])
    gb = _unpack(gb_ref[...][:, HH:])
    l1 = jax.nn.relu(ga + gb + _dot(e_in, w1c[...]) + b1[...])
    l2 = jax.nn.relu(_dot(l1, w2[...]) + b2[...])
    l3 = _dot(l2, w3[...]) + b3[...]
    e_new = _ln(l3, g[...], beta[...]) + e_in
    e_out[...] = e_new
    mm = _dot(e_new, wm[...])
    m_out[0] = mm[:, :HH]
    m_out[1] = mm[:, HH:]


_edge_mlp = pl.pallas_call(
    _edge_mlp_body,
    grid=(E2 // TRH,),
    in_specs=[pl.BlockSpec((TRH, H), lambda i: (i, 0))] * 3
    + _w_specs([(H, H), (1, H), (H, H), (1, H), (H, H), (1, H), (1, H), (1, H),
                (H, H)]),
    out_specs=[pl.BlockSpec((TRH, H), lambda i: (i, 0)),
               pl.BlockSpec((2, TRH, HH), lambda i: (0, i, 0))],
    out_shape=[jax.ShapeDtypeStruct((E2, H), jnp.float32),
               jax.ShapeDtypeStruct((2, E2, HH), jnp.float32)],
)


def _agg_cat(a0, a1):
    return jnp.concatenate([a0[0], a1[0]], axis=-1)


_AGG_SPECS = [pl.BlockSpec((1, TRN, HH), lambda i: (0, i, 0)),
              pl.BlockSpec((1, TRN, HH), lambda i: (1, i, 0))]


def _node_mlp_body(h_ref, a0_ref, a1_ref,
                   w1, b1, w2, b2, w3, b3, g, beta, wa, wb, h_out, hab_out):
    h_in = h_ref[...]
    agg = _agg_cat(a0_ref, a1_ref)
    l1 = jax.nn.relu(_dot(h_in, w1[...]) + agg + b1[...])
    l2 = jax.nn.relu(_dot(l1, w2[...]) + b2[...])
    l3 = _dot(l2, w3[...]) + b3[...]
    h_new = _ln(l3, g[...], beta[...]) + h_in
    h_out[...] = h_new
    hab_out[...] = jnp.concatenate(
        [_pack(_dot(h_new, wa[...])), _pack(_dot(h_new, wb[...]))], axis=-1)


_node_mlp = pl.pallas_call(
    _node_mlp_body,
    grid=(NPAD // TRN,),
    in_specs=[pl.BlockSpec((TRN, H), lambda i: (i, 0))] + _AGG_SPECS
    + _w_specs([(H, H), (1, H), (H, H), (1, H), (H, H), (1, H), (1, H), (1, H),
                (H, H), (H, H)]),
    out_specs=[pl.BlockSpec((TRN, H), lambda i: (i, 0)),
               pl.BlockSpec((TRN, H), lambda i: (i, 0))],
    out_shape=[jax.ShapeDtypeStruct((NPAD, H), jnp.float32),
               jax.ShapeDtypeStruct((NPAD, H), jnp.uint32)],
)


def _node_last_body(h_ref, a0_ref, a1_ref,
                    w1, b1, w2, b2, w3, b3, g, beta, h_out):
    h_in = h_ref[...]
    agg = _agg_cat(a0_ref, a1_ref)
    l1 = jax.nn.relu(_dot(h_in, w1[...]) + agg + b1[...])
    l2 = jax.nn.relu(_dot(l1, w2[...]) + b2[...])
    l3 = _dot(l2, w3[...]) + b3[...]
    h_out[...] = _ln(l3, g[...], beta[...]) + h_in


_node_last = pl.pallas_call(
    _node_last_body,
    grid=(NPAD // TRN,),
    in_specs=[pl.BlockSpec((TRN, H), lambda i: (i, 0))] + _AGG_SPECS
    + _w_specs([(H, H), (1, H), (H, H), (1, H), (H, H), (1, H), (1, H),
                (1, H)]),
    out_specs=pl.BlockSpec((TRN, H), lambda i: (i, 0)),
    out_shape=jax.ShapeDtypeStruct((NPAD, H), jnp.float32),
)


def _dec_body(h_ref, x_ref, w1, b1, w2, b2, w3, b3, o_ref):
    l1 = jax.nn.relu(_dot(h_ref[...], w1[...]) + b1[...])
    l2 = jax.nn.relu(_dot(l1, w2[...]) + b2[...])
    l3 = _dot(l2, w3[...]) + b3[...]
    o_ref[...] = l3 * 0.005 + x_ref[:, :3]


_dec = pl.pallas_call(
    _dec_body,
    grid=(NPAD // TRN,),
    in_specs=[pl.BlockSpec((TRN, H), lambda i: (i, 0)),
              pl.BlockSpec((TRN, H), lambda i: (i, 0))]
    + _w_specs([(H, H), (1, H), (H, H), (1, H), (H, 3), (1, 3)]),
    out_specs=pl.BlockSpec((TRN, 3), lambda i: (i, 0)),
    out_shape=jax.ShapeDtypeStruct((NPAD, 3), jnp.float32),
)


_sc_mesh = plsc.VectorSubcoreMesh(core_axis_name="c", subcore_axis_name="s")


@functools.partial(
    pl.kernel,
    out_type=[jax.ShapeDtypeStruct((E2, H), jnp.uint32),
              jax.ShapeDtypeStruct((E2, H), jnp.uint32)],
    mesh=_sc_mesh,
    scratch_types=[pltpu.VMEM((GSLOT, CH), jnp.int32),
                   pltpu.VMEM((GSLOT, CH, H), jnp.uint32),
                   pltpu.SemaphoreType.DMA((GSLOT,)),
                   pltpu.SemaphoreType.DMA((GSLOT,)),
                   pltpu.SemaphoreType.DMA((GSLOT,))],
)
def _sc_gather(hab_hbm, idxg_hbm, ga_hbm, gb_hbm, idx, buf, isem, gsem, wsem):
    cid = lax.axis_index("c")
    sid = lax.axis_index("s")
    wid = sid * 2 + cid
    base = wid * PW

    for phase in range(2):
        cbase = phase * NCHG
        out = (ga_hbm, gb_hbm)[phase]

        def _wait_write(b, _out=out):
            pltpu.make_async_copy(buf.at[b], _out.at[pl.ds(base, CH)],
                                  wsem.at[b]).wait()

        for b in range(GSLOT):
            pltpu.async_copy(idxg_hbm.at[wid, cbase + b], idx.at[b],
                             isem.at[b])
        for b in range(GLOOK):
            pltpu.make_async_copy(idxg_hbm.at[wid, 0], idx.at[b],
                                  isem.at[b]).wait()
            pltpu.async_copy(hab_hbm.at[idx.at[b]], buf.at[b], gsem.at[b])

        @pl.loop(0, NCHG, step=GSLOT)
        def _round(step, _out=out, _cbase=cbase, _wait_write=_wait_write):
            for b in range(GSLOT):
                v = step + b
                ub = (b + GLOOK) % GSLOT
                pltpu.make_async_copy(hab_hbm.at[idx.at[b]], buf.at[b],
                                      gsem.at[b]).wait()
                pltpu.async_copy(buf.at[b],
                                 _out.at[pl.ds(base + v * CH, CH)],
                                 wsem.at[b])
                cw = v + GSLOT

                @pl.when(cw < NCHG)
                def _():
                    pltpu.async_copy(idxg_hbm.at[wid, _cbase + cw], idx.at[b],
                                     isem.at[b])

                u = v + GLOOK

                @pl.when(u < NCHG)
                def _():
                    @pl.when(u >= GSLOT)
                    def _():
                        _wait_write(ub)

                    pltpu.make_async_copy(idxg_hbm.at[wid, 0], idx.at[ub],
                                          isem.at[ub]).wait()
                    pltpu.async_copy(hab_hbm.at[idx.at[ub]], buf.at[ub],
                                     gsem.at[ub])

        for b in range(GSLOT):
            _wait_write(b)


@functools.partial(
    pl.kernel,
    out_type=jax.ShapeDtypeStruct((2, NPAD, HH), jnp.float32),
    mesh=_sc_mesh,
    scratch_types=[pltpu.VMEM_SHARED((NPAD, HH), jnp.float32),
                   pltpu.VMEM((NSLOT, CHS), jnp.int32),
                   pltpu.VMEM((NSLOT, CHS, HH), jnp.float32),
                   pltpu.SemaphoreType.DMA((NSLOT,)),
                   pltpu.SemaphoreType.DMA((NSLOT,)),
                   pltpu.SemaphoreType.DMA((NSLOT,))],
)
def _sc_scatter(ms_hbm, cols_hbm, zeros_hbm, agg_hbm, shared, idx, buf,
                isem, lsem, asem):
    cid = lax.axis_index("c")
    sid = lax.axis_index("s")
    base = sid * PWS
    pltpu.sync_copy(zeros_hbm.at[pl.ds(sid * RPT, RPT)],
                    shared.at[pl.ds(sid * RPT, RPT)])
    plsc.subcore_barrier()
    for b in range(LOOK):
        pltpu.async_copy(cols_hbm.at[sid, b], idx.at[b], isem.at[b])
        pltpu.async_copy(ms_hbm.at[cid, pl.ds(base + b * CHS, CHS)],
                         buf.at[b], lsem.at[b])

    @pl.loop(0, NCHS, step=NSLOT)
    def _round(step):
        for b in range(NSLOT):
            v = step + b
            ub = (b + LOOK) % NSLOT
            pltpu.make_async_copy(cols_hbm.at[sid, 0], idx.at[b],
                                  isem.at[b]).wait()
            pltpu.make_async_copy(ms_hbm.at[cid, pl.ds(base, CHS)], buf.at[b],
                                  lsem.at[b]).wait()
            pltpu.async_copy(buf.at[b], shared.at[idx.at[b]], asem.at[b],
                             add=True)
            u = v + LOOK

            @pl.when(u < NCHS)
            def _():
                @pl.when(u >= NSLOT)
                def _():
                    pltpu.make_async_copy(buf.at[ub], shared.at[idx.at[ub]],
                                          asem.at[ub]).wait()

                pltpu.async_copy(cols_hbm.at[sid, u], idx.at[ub],
                                 isem.at[ub])
                pltpu.async_copy(ms_hbm.at[cid, pl.ds(base + u * CHS, CHS)],
                                 buf.at[ub], lsem.at[ub])

    for b in range(NSLOT):
        pltpu.make_async_copy(buf.at[b], shared.at[idx.at[b]],
                              asem.at[b]).wait()
    plsc.subcore_barrier()
    pltpu.sync_copy(shared.at[pl.ds(sid * RPT, RPT)],
                    agg_hbm.at[cid, pl.ds(sid * RPT, RPT)])


def _r(v):
    return v.reshape(1, -1)


def kernel(x, edge_index, edge_attr, params):
    row = edge_index[0]
    col = edge_index[1]
    idxg = jnp.concatenate([row.reshape(NW, NCHG, CH),
                            col.reshape(NW, NCHG, CH)], axis=1)
    cols = col.reshape(16, NCHS, CHS)
    xp = jnp.pad(x, ((0, NPAD - N), (0, 0)))
    zeros_n = jnp.zeros((NPAD, HH), jnp.float32)

    sums = _bn_stats(edge_attr)
    mu = sums[0] / E
    var = sums[1] / E - mu * mu
    s = params["bn"]["gamma"] * lax.rsqrt(var + 1e-5)
    ep = params["edge_enc"]
    w1p = ep["W"][0] * s[:, None]
    b1p = ep["b"][0] + (params["bn"]["beta"] - mu * s) @ ep["W"][0]
    e = _enc_edge(edge_attr, w1p, _r(b1p), ep["W"][1], _r(ep["b"][1]),
                  ep["W"][2], _r(ep["b"][2]), _r(ep["g"]), _r(ep["beta"]))

    np_ = params["node_enc"]
    we0 = params["blocks"][0]["edge"]["W"][0]
    h, hab = _enc_node(xp, np_["W"][0], _r(np_["b"][0]),
                       np_["W"][1], _r(np_["b"][1]),
                       np_["W"][2], _r(np_["b"][2]),
                       _r(np_["g"]), _r(np_["beta"]),
                       we0[:H], we0[H:2 * H])

    for i in range(MP):
        blk = params["blocks"][i]
        pe, pn = blk["edge"], blk["node"]
        eargs = (pe["W"][0][2 * H:], _r(pe["b"][0]),
                 pe["W"][1], _r(pe["b"][1]), pe["W"][2], _r(pe["b"][2]),
                 _r(pe["g"]), _r(pe["beta"]), pn["W"][0][H:])
        ga, gb = _sc_gather(hab, idxg)
        e, ms = _edge_mlp(ga, gb, e, *eargs)
        agg2 = _sc_scatter(ms, cols, zeros_n)
        nargs = (h, agg2, agg2,
                 pn["W"][0][:H], _r(pn["b"][0]),
                 pn["W"][1], _r(pn["b"][1]), pn["W"][2], _r(pn["b"][2]),
                 _r(pn["g"]), _r(pn["beta"]))
        if i < MP - 1:
            wen = params["blocks"][i + 1]["edge"]["W"][0]
            h, hab = _node_mlp(*nargs, wen[:H], wen[H:2 * H])
        else:
            h = _node_last(*nargs)

    d = params["dec"]
    out = _dec(h, xp, d["W"][0], _r(d["b"][0]), d["W"][1], _r(d["b"][1]),
               d["W"][2], _r(d["b"][2]))
    return out[:N]


# speedup vs baseline: 1.0712x; 1.0382x over previous
import functools

import jax
import jax.numpy as jnp
from jax import lax
from jax.experimental import pallas as pl
from jax.experimental.pallas import tpu as pltpu
from jax.experimental.pallas import tpu_sc as plsc

N = 10000
NPAD = 10240
E = 320000
H = 128
HH = H // 2
D_EDGE = 16
MP = 5

TRE = 512
TRN = 512

NW = 32
PW = E // NW
CH = 40
NCH = PW // CH
PWS = E // 16
CHS = 80
NCHS = PWS // CHS
NB = 5
RPT = NPAD // 16


def _ln(xx, g, beta):
    m = jnp.mean(xx, axis=-1, keepdims=True)
    v = jnp.mean((xx - m) ** 2, axis=-1, keepdims=True)
    return (xx - m) * lax.rsqrt(v + 1e-5) * g + beta


def _dot(a, b):
    return jnp.dot(a, b, preferred_element_type=jnp.float32)


def _bn_stats_body(x_ref, o_ref):
    i = pl.program_id(0)
    xb = x_ref[...]
    part = jnp.concatenate(
        [jnp.sum(xb, axis=0, keepdims=True),
         jnp.sum(xb * xb, axis=0, keepdims=True)], axis=0)

    @pl.when(i == 0)
    def _():
        o_ref[...] = part

    @pl.when(i > 0)
    def _():
        o_ref[...] += part


_bn_stats = pl.pallas_call(
    _bn_stats_body,
    grid=(E // TRE,),
    in_specs=[pl.BlockSpec((TRE, D_EDGE), lambda i: (i, 0))],
    out_specs=pl.BlockSpec((2, D_EDGE), lambda i: (0, 0)),
    out_shape=jax.ShapeDtypeStruct((2, D_EDGE), jnp.float32),
)


def _w_specs(shapes):
    return [pl.BlockSpec(s, lambda i: tuple(0 for _ in s)) for s in shapes]


def _enc_edge_body(x_ref, w1, b1, w2, b2, w3, b3, g, beta, o_ref):
    l1 = jax.nn.relu(_dot(x_ref[...], w1[...]) + b1[...])
    l2 = jax.nn.relu(_dot(l1, w2[...]) + b2[...])
    l3 = _dot(l2, w3[...]) + b3[...]
    o_ref[...] = _ln(l3, g[...], beta[...])


_enc_edge = pl.pallas_call(
    _enc_edge_body,
    grid=(E // TRE,),
    in_specs=[pl.BlockSpec((TRE, D_EDGE), lambda i: (i, 0))]
    + _w_specs([(D_EDGE, H), (1, H), (H, H), (1, H), (H, H), (1, H), (1, H),
                (1, H)]),
    out_specs=pl.BlockSpec((TRE, H), lambda i: (i, 0)),
    out_shape=jax.ShapeDtypeStruct((E, H), jnp.float32),
)


def _enc_node_body(x_ref, w1, b1, w2, b2, w3, b3, g, beta, wa, wb,
                   h_ref, ha_ref, hb_ref):
    l1 = jax.nn.relu(_dot(x_ref[...], w1[...]) + b1[...])
    l2 = jax.nn.relu(_dot(l1, w2[...]) + b2[...])
    l3 = _dot(l2, w3[...]) + b3[...]
    h = _ln(l3, g[...], beta[...])
    h_ref[...] = h
    ha_ref[...] = _dot(h, wa[...])
    hb_ref[...] = _dot(h, wb[...])


_enc_node = pl.pallas_call(
    _enc_node_body,
    grid=(NPAD // TRN,),
    in_specs=[pl.BlockSpec((TRN, H), lambda i: (i, 0))]
    + _w_specs([(H, H), (1, H), (H, H), (1, H), (H, H), (1, H), (1, H), (1, H),
                (H, H), (H, H)]),
    out_specs=[pl.BlockSpec((TRN, H), lambda i: (i, 0))] * 3,
    out_shape=[jax.ShapeDtypeStruct((NPAD, H), jnp.float32)] * 3,
)


def _edge_mlp_body(ga_ref, gb_ref, e_ref, w1c, b1, w2, b2, w3, b3, g, beta, wm,
                   e_out, m_out):
    e_in = e_ref[...]
    l1 = jax.nn.relu(ga_ref[...] + gb_ref[...] + _dot(e_in, w1c[...]) + b1[...])
    l2 = jax.nn.relu(_dot(l1, w2[...]) + b2[...])
    l3 = _dot(l2, w3[...]) + b3[...]
    e_new = _ln(l3, g[...], beta[...]) + e_in
    e_out[...] = e_new
    mm = _dot(e_new, wm[...])
    m_out[0] = mm[:, :HH]
    m_out[1] = mm[:, HH:]


_edge_mlp = pl.pallas_call(
    _edge_mlp_body,
    grid=(E // TRE,),
    in_specs=[pl.BlockSpec((TRE, H), lambda i: (i, 0))] * 3
    + _w_specs([(H, H), (1, H), (H, H), (1, H), (H, H), (1, H), (1, H), (1, H),
                (H, H)]),
    out_specs=[pl.BlockSpec((TRE, H), lambda i: (i, 0)),
               pl.BlockSpec((2, TRE, HH), lambda i: (0, i, 0))],
    out_shape=[jax.ShapeDtypeStruct((E, H), jnp.float32),
               jax.ShapeDtypeStruct((2, E, HH), jnp.float32)],
)


_AGG_SPECS = [pl.BlockSpec((1, TRN, HH), lambda i: (0, i, 0)),
              pl.BlockSpec((1, TRN, HH), lambda i: (1, i, 0))]


def _node_mlp_body(h_ref, a0_ref, a1_ref, w1, b1, w2, b2, w3, b3, g, beta,
                   wa, wb, h_out, ha_out, hb_out):
    h_in = h_ref[...]
    agg = jnp.concatenate([a0_ref[0], a1_ref[0]], axis=-1)
    l1 = jax.nn.relu(_dot(h_in, w1[...]) + agg + b1[...])
    l2 = jax.nn.relu(_dot(l1, w2[...]) + b2[...])
    l3 = _dot(l2, w3[...]) + b3[...]
    h_new = _ln(l3, g[...], beta[...]) + h_in
    h_out[...] = h_new
    ha_out[...] = _dot(h_new, wa[...])
    hb_out[...] = _dot(h_new, wb[...])


_node_mlp = pl.pallas_call(
    _node_mlp_body,
    grid=(NPAD // TRN,),
    in_specs=[pl.BlockSpec((TRN, H), lambda i: (i, 0))] + _AGG_SPECS
    + _w_specs([(H, H), (1, H), (H, H), (1, H), (H, H), (1, H), (1, H), (1, H),
                (H, H), (H, H)]),
    out_specs=[pl.BlockSpec((TRN, H), lambda i: (i, 0))] * 3,
    out_shape=[jax.ShapeDtypeStruct((NPAD, H), jnp.float32)] * 3,
)


def _node_last_body(h_ref, a0_ref, a1_ref, w1, b1, w2, b2, w3, b3, g, beta,
                    h_out):
    h_in = h_ref[...]
    agg = jnp.concatenate([a0_ref[0], a1_ref[0]], axis=-1)
    l1 = jax.nn.relu(_dot(h_in, w1[...]) + agg + b1[...])
    l2 = jax.nn.relu(_dot(l1, w2[...]) + b2[...])
    l3 = _dot(l2, w3[...]) + b3[...]
    h_out[...] = _ln(l3, g[...], beta[...]) + h_in


_node_last = pl.pallas_call(
    _node_last_body,
    grid=(NPAD // TRN,),
    in_specs=[pl.BlockSpec((TRN, H), lambda i: (i, 0))] + _AGG_SPECS
    + _w_specs([(H, H), (1, H), (H, H), (1, H), (H, H), (1, H), (1, H),
                (1, H)]),
    out_specs=pl.BlockSpec((TRN, H), lambda i: (i, 0)),
    out_shape=jax.ShapeDtypeStruct((NPAD, H), jnp.float32),
)


def _dec_body(h_ref, x_ref, w1, b1, w2, b2, w3, b3, o_ref):
    l1 = jax.nn.relu(_dot(h_ref[...], w1[...]) + b1[...])
    l2 = jax.nn.relu(_dot(l1, w2[...]) + b2[...])
    l3 = _dot(l2, w3[...]) + b3[...]
    o_ref[...] = l3 * 0.005 + x_ref[:, :3]


_dec = pl.pallas_call(
    _dec_body,
    grid=(NPAD // TRN,),
    in_specs=[pl.BlockSpec((TRN, H), lambda i: (i, 0)),
              pl.BlockSpec((TRN, H), lambda i: (i, 0))]
    + _w_specs([(H, H), (1, H), (H, H), (1, H), (H, 3), (1, 3)]),
    out_specs=pl.BlockSpec((TRN, 3), lambda i: (i, 0)),
    out_shape=jax.ShapeDtypeStruct((NPAD, 3), jnp.float32),
)


_sc_mesh = plsc.VectorSubcoreMesh(core_axis_name="c", subcore_axis_name="s")


@functools.partial(
    pl.kernel,
    out_type=[jax.ShapeDtypeStruct((E, H), jnp.float32),
              jax.ShapeDtypeStruct((E, H), jnp.float32)],
    mesh=_sc_mesh,
    scratch_types=[pltpu.VMEM((NCH, CH), jnp.int32),
                   pltpu.VMEM((NCH, CH), jnp.int32),
                   pltpu.VMEM((NB, CH, H), jnp.float32),
                   pltpu.VMEM((NB, CH, H), jnp.float32),
                   pltpu.SemaphoreType.DMA((NB,)),
                   pltpu.SemaphoreType.DMA((NB,)),
                   pltpu.SemaphoreType.DMA((NB,)),
                   pltpu.SemaphoreType.DMA((NB,))],
)
def _sc_gather(ha_hbm, hb_hbm, row3_hbm, col3_hbm, ga_hbm, gb_hbm,
               idx_a, idx_b, buf_a, buf_b, gsa, gsb, wsa, wsb):
    cid = lax.axis_index("c")
    sid = lax.axis_index("s")
    wid = sid * 2 + cid
    base = wid * PW
    pltpu.sync_copy(row3_hbm.at[wid], idx_a)
    pltpu.sync_copy(col3_hbm.at[wid], idx_b)
    for b in range(NB):
        pltpu.async_copy(ha_hbm.at[idx_a.at[b]], buf_a.at[b], gsa.at[b])
        pltpu.async_copy(hb_hbm.at[idx_b.at[b]], buf_b.at[b], gsb.at[b])

    @pl.loop(0, NCH, step=NB)
    def _round(step):
        for b in range(NB):
            ci = step + b
            s = base + ci * CH
            pltpu.make_async_copy(ha_hbm.at[idx_a.at[ci]], buf_a.at[b],
                                  gsa.at[b]).wait()
            pltpu.make_async_copy(hb_hbm.at[idx_b.at[ci]], buf_b.at[b],
                                  gsb.at[b]).wait()
            pltpu.async_copy(buf_a.at[b], ga_hbm.at[pl.ds(s, CH)], wsa.at[b])
            pltpu.async_copy(buf_b.at[b], gb_hbm.at[pl.ds(s, CH)], wsb.at[b])
        for b in range(NB):
            cj = step + NB + b
            pltpu.make_async_copy(buf_a.at[b], ga_hbm.at[pl.ds(base, CH)],
                                  wsa.at[b]).wait()
            pltpu.make_async_copy(buf_b.at[b], gb_hbm.at[pl.ds(base, CH)],
                                  wsb.at[b]).wait()

            @pl.when(cj < NCH)
            def _():
                pltpu.async_copy(ha_hbm.at[idx_a.at[cj]], buf_a.at[b],
                                 gsa.at[b])
                pltpu.async_copy(hb_hbm.at[idx_b.at[cj]], buf_b.at[b],
                                 gsb.at[b])


@functools.partial(
    pl.kernel,
    out_type=jax.ShapeDtypeStruct((2, NPAD, HH), jnp.float32),
    mesh=_sc_mesh,
    scratch_types=[pltpu.VMEM_SHARED((NPAD, HH), jnp.float32),
                   pltpu.VMEM((NCHS, CHS), jnp.int32),
                   pltpu.VMEM((NB, CHS, HH), jnp.float32),
                   pltpu.SemaphoreType.DMA((NB,))],
)
def _sc_scatter(ms_hbm, cols_hbm, zeros_hbm, agg_hbm, shared, idx, buf, lsem):
    cid = lax.axis_index("c")
    sid = lax.axis_index("s")
    base = sid * PWS
    pltpu.sync_copy(zeros_hbm.at[pl.ds(sid * RPT, RPT)],
                    shared.at[pl.ds(sid * RPT, RPT)])
    pltpu.sync_copy(cols_hbm.at[sid], idx)
    plsc.subcore_barrier()
    for b in range(NB):
        pltpu.async_copy(ms_hbm.at[cid, pl.ds(base + b * CHS, CHS)],
                         buf.at[b], lsem.at[b])

    @pl.loop(0, NCHS, step=NB)
    def _round(step):
        for b in range(NB):
            ci = step + b
            cj = ci + NB
            pltpu.make_async_copy(ms_hbm.at[cid, pl.ds(base, CHS)], buf.at[b],
                                  lsem.at[b]).wait()
            pltpu.sync_copy(buf.at[b], shared.at[idx.at[ci]], add=True)

            @pl.when(cj < NCHS)
            def _():
                pltpu.async_copy(ms_hbm.at[cid, pl.ds(base + cj * CHS, CHS)],
                                 buf.at[b], lsem.at[b])

    plsc.subcore_barrier()
    pltpu.sync_copy(shared.at[pl.ds(sid * RPT, RPT)],
                    agg_hbm.at[cid, pl.ds(sid * RPT, RPT)])


def _r(v):
    return v.reshape(1, -1)


def kernel(x, edge_index, edge_attr, params):
    row3 = edge_index[0].reshape(NW, NCH, CH)
    col3 = edge_index[1].reshape(NW, NCH, CH)
    cols = edge_index[1].reshape(16, NCHS, CHS)
    xp = jnp.pad(x, ((0, NPAD - N), (0, 0)))
    zeros_n = jnp.zeros((NPAD, HH), jnp.float32)

    sums = _bn_stats(edge_attr)
    mu = sums[0] / E
    var = sums[1] / E - mu * mu
    s = params["bn"]["gamma"] * lax.rsqrt(var + 1e-5)
    ep = params["edge_enc"]
    w1p = ep["W"][0] * s[:, None]
    b1p = ep["b"][0] + (params["bn"]["beta"] - mu * s) @ ep["W"][0]
    e = _enc_edge(edge_attr, w1p, _r(b1p), ep["W"][1], _r(ep["b"][1]),
                  ep["W"][2], _r(ep["b"][2]), _r(ep["g"]), _r(ep["beta"]))

    np_ = params["node_enc"]
    we0 = params["blocks"][0]["edge"]["W"][0]
    h, ha, hb = _enc_node(xp, np_["W"][0], _r(np_["b"][0]),
                          np_["W"][1], _r(np_["b"][1]),
                          np_["W"][2], _r(np_["b"][2]),
                          _r(np_["g"]), _r(np_["beta"]),
                          we0[:H], we0[H:2 * H])

    for i in range(MP):
        blk = params["blocks"][i]
        pe, pn = blk["edge"], blk["node"]
        ga, gb = _sc_gather(ha, hb, row3, col3)
        e, ms = _edge_mlp(ga, gb, e, pe["W"][0][2 * H:], _r(pe["b"][0]),
                          pe["W"][1], _r(pe["b"][1]), pe["W"][2],
                          _r(pe["b"][2]), _r(pe["g"]), _r(pe["beta"]),
                          pn["W"][0][H:])
        agg2 = _sc_scatter(ms, cols, zeros_n)
        nargs = (h, agg2, agg2, pn["W"][0][:H], _r(pn["b"][0]),
                 pn["W"][1], _r(pn["b"][1]), pn["W"][2], _r(pn["b"][2]),
                 _r(pn["g"]), _r(pn["beta"]))
        if i < MP - 1:
            wen = params["blocks"][i + 1]["edge"]["W"][0]
            h, ha, hb = _node_mlp(*nargs, wen[:H], wen[H:2 * H])
        else:
            h = _node_last(*nargs)

    d = params["dec"]
    out = _dec(h, xp, d["W"][0], _r(d["b"][0]), d["W"][1], _r(d["b"][1]),
               d["W"][2], _r(d["b"][2]))
    return out[:N]


# speedup vs baseline: 1.2914x; 1.2056x over previous
import functools

import jax
import jax.numpy as jnp
from jax import lax
from jax.experimental import pallas as pl
from jax.experimental.pallas import tpu as pltpu
from jax.experimental.pallas import tpu_sc as plsc

N = 10000
NPAD = 10240
E = 320000
H = 128
HH = H // 2
D_EDGE = 16
MP = 5

TRE = 800
TRN = 512

NW = 32
PW = E // NW
CH = 40
NCH = PW // CH
PWS = E // 16
CHS = 80
NCHS = PWS // CHS
NB = 5
RPT = NPAD // 16


def _ln(xx, g, beta):
    m = jnp.mean(xx, axis=-1, keepdims=True)
    v = jnp.mean((xx - m) ** 2, axis=-1, keepdims=True)
    return (xx - m) * lax.rsqrt(v + 1e-5) * g + beta


def _dot(a, b):
    return jnp.dot(a, b, preferred_element_type=jnp.float32)


def _bn_stats_body(x_ref, o_ref):
    i = pl.program_id(0)
    xb = x_ref[...]
    part = jnp.concatenate(
        [jnp.sum(xb, axis=0, keepdims=True),
         jnp.sum(xb * xb, axis=0, keepdims=True)], axis=0)

    @pl.when(i == 0)
    def _():
        o_ref[...] = part

    @pl.when(i > 0)
    def _():
        o_ref[...] += part


_bn_stats = pl.pallas_call(
    _bn_stats_body,
    grid=(E // TRE,),
    in_specs=[pl.BlockSpec((TRE, D_EDGE), lambda i: (i, 0))],
    out_specs=pl.BlockSpec((2, D_EDGE), lambda i: (0, 0)),
    out_shape=jax.ShapeDtypeStruct((2, D_EDGE), jnp.float32),
)


def _w_specs(shapes):
    return [pl.BlockSpec(s, lambda i: tuple(0 for _ in s)) for s in shapes]


def _enc_edge_body(x_ref, w1, b1, w2, b2, w3, b3, g, beta, o_ref):
    l1 = jax.nn.relu(_dot(x_ref[...], w1[...]) + b1[...])
    l2 = jax.nn.relu(_dot(l1, w2[...]) + b2[...])
    l3 = _dot(l2, w3[...]) + b3[...]
    o_ref[...] = _ln(l3, g[...], beta[...])


_enc_edge = pl.pallas_call(
    _enc_edge_body,
    grid=(E // TRE,),
    in_specs=[pl.BlockSpec((TRE, D_EDGE), lambda i: (i, 0))]
    + _w_specs([(D_EDGE, H), (1, H), (H, H), (1, H), (H, H), (1, H), (1, H),
                (1, H)]),
    out_specs=pl.BlockSpec((TRE, H), lambda i: (i, 0)),
    out_shape=jax.ShapeDtypeStruct((E, H), jnp.float32),
)


def _enc_node_body(x_ref, w1, b1, w2, b2, w3, b3, g, beta, wa, wb,
                   h_ref, ha_ref, hb_ref):
    l1 = jax.nn.relu(_dot(x_ref[...], w1[...]) + b1[...])
    l2 = jax.nn.relu(_dot(l1, w2[...]) + b2[...])
    l3 = _dot(l2, w3[...]) + b3[...]
    h = _ln(l3, g[...], beta[...])
    h_ref[...] = h
    ha_ref[...] = _dot(h, wa[...])
    hb_ref[...] = _dot(h, wb[...])


_enc_node = pl.pallas_call(
    _enc_node_body,
    grid=(NPAD // TRN,),
    in_specs=[pl.BlockSpec((TRN, H), lambda i: (i, 0))]
    + _w_specs([(H, H), (1, H), (H, H), (1, H), (H, H), (1, H), (1, H), (1, H),
                (H, H), (H, H)]),
    out_specs=[pl.BlockSpec((TRN, H), lambda i: (i, 0))] * 3,
    out_shape=[jax.ShapeDtypeStruct((NPAD, H), jnp.float32)] * 3,
)


def _edge_mlp_body(ga_ref, gb_ref, e_ref, w1c, b1, w2, b2, w3, b3, g, beta, wm,
                   e_out, m_out):
    e_in = e_ref[...]
    l1 = jax.nn.relu(ga_ref[...] + gb_ref[...] + _dot(e_in, w1c[...]) + b1[...])
    l2 = jax.nn.relu(_dot(l1, w2[...]) + b2[...])
    l3 = _dot(l2, w3[...]) + b3[...]
    e_new = _ln(l3, g[...], beta[...]) + e_in
    e_out[...] = e_new
    mm = _dot(e_new, wm[...])
    m_out[0] = mm[:, :HH]
    m_out[1] = mm[:, HH:]


_edge_mlp = pl.pallas_call(
    _edge_mlp_body,
    grid=(E // TRE,),
    in_specs=[pl.BlockSpec((TRE, H), lambda i: (i, 0))] * 3
    + _w_specs([(H, H), (1, H), (H, H), (1, H), (H, H), (1, H), (1, H), (1, H),
                (H, H)]),
    out_specs=[pl.BlockSpec((TRE, H), lambda i: (i, 0)),
               pl.BlockSpec((2, TRE, HH), lambda i: (0, i, 0))],
    out_shape=[jax.ShapeDtypeStruct((E, H), jnp.float32),
               jax.ShapeDtypeStruct((2, E, HH), jnp.float32)],
)


_AGG_SPECS = [pl.BlockSpec((1, TRN, HH), lambda i: (0, i, 0)),
              pl.BlockSpec((1, TRN, HH), lambda i: (1, i, 0))]


def _node_mlp_body(h_ref, a0_ref, a1_ref, w1, b1, w2, b2, w3, b3, g, beta,
                   wa, wb, h_out, ha_out, hb_out):
    h_in = h_ref[...]
    agg = jnp.concatenate([a0_ref[0], a1_ref[0]], axis=-1)
    l1 = jax.nn.relu(_dot(h_in, w1[...]) + agg + b1[...])
    l2 = jax.nn.relu(_dot(l1, w2[...]) + b2[...])
    l3 = _dot(l2, w3[...]) + b3[...]
    h_new = _ln(l3, g[...], beta[...]) + h_in
    h_out[...] = h_new
    ha_out[...] = _dot(h_new, wa[...])
    hb_out[...] = _dot(h_new, wb[...])


_node_mlp = pl.pallas_call(
    _node_mlp_body,
    grid=(NPAD // TRN,),
    in_specs=[pl.BlockSpec((TRN, H), lambda i: (i, 0))] + _AGG_SPECS
    + _w_specs([(H, H), (1, H), (H, H), (1, H), (H, H), (1, H), (1, H), (1, H),
                (H, H), (H, H)]),
    out_specs=[pl.BlockSpec((TRN, H), lambda i: (i, 0))] * 3,
    out_shape=[jax.ShapeDtypeStruct((NPAD, H), jnp.float32)] * 3,
)


def _node_last_body(h_ref, a0_ref, a1_ref, w1, b1, w2, b2, w3, b3, g, beta,
                    h_out):
    h_in = h_ref[...]
    agg = jnp.concatenate([a0_ref[0], a1_ref[0]], axis=-1)
    l1 = jax.nn.relu(_dot(h_in, w1[...]) + agg + b1[...])
    l2 = jax.nn.relu(_dot(l1, w2[...]) + b2[...])
    l3 = _dot(l2, w3[...]) + b3[...]
    h_out[...] = _ln(l3, g[...], beta[...]) + h_in


_node_last = pl.pallas_call(
    _node_last_body,
    grid=(NPAD // TRN,),
    in_specs=[pl.BlockSpec((TRN, H), lambda i: (i, 0))] + _AGG_SPECS
    + _w_specs([(H, H), (1, H), (H, H), (1, H), (H, H), (1, H), (1, H),
                (1, H)]),
    out_specs=pl.BlockSpec((TRN, H), lambda i: (i, 0)),
    out_shape=jax.ShapeDtypeStruct((NPAD, H), jnp.float32),
)


def _dec_body(h_ref, x_ref, w1, b1, w2, b2, w3, b3, o_ref):
    l1 = jax.nn.relu(_dot(h_ref[...], w1[...]) + b1[...])
    l2 = jax.nn.relu(_dot(l1, w2[...]) + b2[...])
    l3 = _dot(l2, w3[...]) + b3[...]
    o_ref[...] = l3 * 0.005 + x_ref[:, :3]


_dec = pl.pallas_call(
    _dec_body,
    grid=(NPAD // TRN,),
    in_specs=[pl.BlockSpec((TRN, H), lambda i: (i, 0)),
              pl.BlockSpec((TRN, H), lambda i: (i, 0))]
    + _w_specs([(H, H), (1, H), (H, H), (1, H), (H, 3), (1, 3)]),
    out_specs=pl.BlockSpec((TRN, 3), lambda i: (i, 0)),
    out_shape=jax.ShapeDtypeStruct((NPAD, 3), jnp.float32),
)


_sc_mesh = plsc.VectorSubcoreMesh(core_axis_name="c", subcore_axis_name="s")


@functools.partial(
    pl.kernel,
    out_type=[jax.ShapeDtypeStruct((E, H), jnp.float32),
              jax.ShapeDtypeStruct((E, H), jnp.float32)],
    mesh=_sc_mesh,
    scratch_types=[pltpu.VMEM((NCH, CH), jnp.int32),
                   pltpu.VMEM((NCH, CH), jnp.int32),
                   pltpu.VMEM((NB, CH, H), jnp.float32),
                   pltpu.VMEM((NB, CH, H), jnp.float32),
                   pltpu.SemaphoreType.DMA((NB,)),
                   pltpu.SemaphoreType.DMA((NB,)),
                   pltpu.SemaphoreType.DMA((NB,)),
                   pltpu.SemaphoreType.DMA((NB,))],
)
def _sc_gather(ha_hbm, hb_hbm, row3_hbm, col3_hbm, ga_hbm, gb_hbm,
               idx_a, idx_b, buf_a, buf_b, gsa, gsb, wsa, wsb):
    cid = lax.axis_index("c")
    sid = lax.axis_index("s")
    wid = sid * 2 + cid
    base = wid * PW
    pltpu.sync_copy(row3_hbm.at[wid], idx_a)
    pltpu.sync_copy(col3_hbm.at[wid], idx_b)
    for b in range(NB):
        pltpu.async_copy(ha_hbm.at[idx_a.at[b]], buf_a.at[b], gsa.at[b])
        pltpu.async_copy(hb_hbm.at[idx_b.at[b]], buf_b.at[b], gsb.at[b])

    @pl.loop(0, NCH, step=NB)
    def _round(step):
        for b in range(NB):
            ci = step + b
            s = base + ci * CH
            pltpu.make_async_copy(ha_hbm.at[idx_a.at[ci]], buf_a.at[b],
                                  gsa.at[b]).wait()
            pltpu.make_async_copy(hb_hbm.at[idx_b.at[ci]], buf_b.at[b],
                                  gsb.at[b]).wait()
            pltpu.async_copy(buf_a.at[b], ga_hbm.at[pl.ds(s, CH)], wsa.at[b])
            pltpu.async_copy(buf_b.at[b], gb_hbm.at[pl.ds(s, CH)], wsb.at[b])
        for b in range(NB):
            cj = step + NB + b
            pltpu.make_async_copy(buf_a.at[b], ga_hbm.at[pl.ds(base, CH)],
                                  wsa.at[b]).wait()
            pltpu.make_async_copy(buf_b.at[b], gb_hbm.at[pl.ds(base, CH)],
                                  wsb.at[b]).wait()

            @pl.when(cj < NCH)
            def _():
                pltpu.async_copy(ha_hbm.at[idx_a.at[cj]], buf_a.at[b],
                                 gsa.at[b])
                pltpu.async_copy(hb_hbm.at[idx_b.at[cj]], buf_b.at[b],
                                 gsb.at[b])


@functools.partial(
    pl.kernel,
    out_type=jax.ShapeDtypeStruct((2, NPAD, HH), jnp.float32),
    mesh=_sc_mesh,
    scratch_types=[pltpu.VMEM_SHARED((NPAD, HH), jnp.float32),
                   pltpu.VMEM((NCHS, CHS), jnp.int32),
                   pltpu.VMEM((NB, CHS, HH), jnp.float32),
                   pltpu.SemaphoreType.DMA((NB,))],
)
def _sc_scatter(ms_hbm, cols_hbm, zeros_hbm, agg_hbm, shared, idx, buf, lsem):
    cid = lax.axis_index("c")
    sid = lax.axis_index("s")
    base = sid * PWS
    pltpu.sync_copy(zeros_hbm.at[pl.ds(sid * RPT, RPT)],
                    shared.at[pl.ds(sid * RPT, RPT)])
    pltpu.sync_copy(cols_hbm.at[sid], idx)
    plsc.subcore_barrier()
    for b in range(NB):
        pltpu.async_copy(ms_hbm.at[cid, pl.ds(base + b * CHS, CHS)],
                         buf.at[b], lsem.at[b])

    @pl.loop(0, NCHS, step=NB)
    def _round(step):
        for b in range(NB):
            ci = step + b
            cj = ci + NB
            pltpu.make_async_copy(ms_hbm.at[cid, pl.ds(base, CHS)], buf.at[b],
                                  lsem.at[b]).wait()
            pltpu.sync_copy(buf.at[b], shared.at[idx.at[ci]], add=True)

            @pl.when(cj < NCHS)
            def _():
                pltpu.async_copy(ms_hbm.at[cid, pl.ds(base + cj * CHS, CHS)],
                                 buf.at[b], lsem.at[b])

    plsc.subcore_barrier()
    pltpu.sync_copy(shared.at[pl.ds(sid * RPT, RPT)],
                    agg_hbm.at[cid, pl.ds(sid * RPT, RPT)])


def _r(v):
    return v.reshape(1, -1)


def kernel(x, edge_index, edge_attr, params):
    row3 = edge_index[0].reshape(NW, NCH, CH)
    col3 = edge_index[1].reshape(NW, NCH, CH)
    cols = edge_index[1].reshape(16, NCHS, CHS)
    xp = jnp.pad(x, ((0, NPAD - N), (0, 0)))
    zeros_n = jnp.zeros((NPAD, HH), jnp.float32)

    sums = _bn_stats(edge_attr)
    mu = sums[0] / E
    var = sums[1] / E - mu * mu
    s = params["bn"]["gamma"] * lax.rsqrt(var + 1e-5)
    ep = params["edge_enc"]
    w1p = ep["W"][0] * s[:, None]
    b1p = ep["b"][0] + (params["bn"]["beta"] - mu * s) @ ep["W"][0]
    e = _enc_edge(edge_attr, w1p, _r(b1p), ep["W"][1], _r(ep["b"][1]),
                  ep["W"][2], _r(ep["b"][2]), _r(ep["g"]), _r(ep["beta"]))

    np_ = params["node_enc"]
    we0 = params["blocks"][0]["edge"]["W"][0]
    h, ha, hb = _enc_node(xp, np_["W"][0], _r(np_["b"][0]),
                          np_["W"][1], _r(np_["b"][1]),
                          np_["W"][2], _r(np_["b"][2]),
                          _r(np_["g"]), _r(np_["beta"]),
                          we0[:H], we0[H:2 * H])

    for i in range(MP):
        blk = params["blocks"][i]
        pe, pn = blk["edge"], blk["node"]
        ga, gb = _sc_gather(ha, hb, row3, col3)
        e, ms = _edge_mlp(ga, gb, e, pe["W"][0][2 * H:], _r(pe["b"][0]),
                          pe["W"][1], _r(pe["b"][1]), pe["W"][2],
                          _r(pe["b"][2]), _r(pe["g"]), _r(pe["beta"]),
                          pn["W"][0][H:])
        agg2 = _sc_scatter(ms, cols, zeros_n)
        nargs = (h, agg2, agg2, pn["W"][0][:H], _r(pn["b"][0]),
                 pn["W"][1], _r(pn["b"][1]), pn["W"][2], _r(pn["b"][2]),
                 _r(pn["g"]), _r(pn["beta"]))
        if i < MP - 1:
            wen = params["blocks"][i + 1]["edge"]["W"][0]
            h, ha, hb = _node_mlp(*nargs, wen[:H], wen[H:2 * H])
        else:
            h = _node_last(*nargs)

    d = params["dec"]
    out = _dec(h, xp, d["W"][0], _r(d["b"][0]), d["W"][1], _r(d["b"][1]),
               d["W"][2], _r(d["b"][2]))
    return out[:N]


# speedup vs baseline: 1.5824x; 1.2254x over previous
import functools

import jax
import jax.numpy as jnp
from jax import lax
from jax.experimental import pallas as pl
from jax.experimental.pallas import tpu as pltpu
from jax.experimental.pallas import tpu_sc as plsc

N = 10000
NPAD = 10240
E = 320000
H = 128
HH = H // 2
D_EDGE = 16
MP = 5

TRE = 1600
TRN = 1024

NW = 32
PW = E // NW
CH = 40
NCH = PW // CH
PWS = E // 16
CHS = 80
NCHS = PWS // CHS
NB = 5
RPT = NPAD // 16


def _ln(xx, g, beta):
    m = jnp.mean(xx, axis=-1, keepdims=True)
    v = jnp.mean((xx - m) ** 2, axis=-1, keepdims=True)
    return (xx - m) * lax.rsqrt(v + 1e-5) * g + beta


def _dot(a, b):
    return jnp.dot(a, b, preferred_element_type=jnp.float32)


def _bn_stats_body(x_ref, o_ref):
    i = pl.program_id(0)
    xb = x_ref[...]
    part = jnp.concatenate(
        [jnp.sum(xb, axis=0, keepdims=True),
         jnp.sum(xb * xb, axis=0, keepdims=True)], axis=0)

    @pl.when(i == 0)
    def _():
        o_ref[...] = part

    @pl.when(i > 0)
    def _():
        o_ref[...] += part


_bn_stats = pl.pallas_call(
    _bn_stats_body,
    grid=(E // TRE,),
    in_specs=[pl.BlockSpec((TRE, D_EDGE), lambda i: (i, 0))],
    out_specs=pl.BlockSpec((2, D_EDGE), lambda i: (0, 0)),
    out_shape=jax.ShapeDtypeStruct((2, D_EDGE), jnp.float32),
)


def _w_specs(shapes):
    return [pl.BlockSpec(s, lambda i: tuple(0 for _ in s)) for s in shapes]


def _enc_edge_body(x_ref, w1, b1, w2, b2, w3, b3, g, beta, o_ref):
    l1 = jax.nn.relu(_dot(x_ref[...], w1[...]) + b1[...])
    l2 = jax.nn.relu(_dot(l1, w2[...]) + b2[...])
    l3 = _dot(l2, w3[...]) + b3[...]
    o_ref[...] = _ln(l3, g[...], beta[...])


_enc_edge = pl.pallas_call(
    _enc_edge_body,
    grid=(E // TRE,),
    in_specs=[pl.BlockSpec((TRE, D_EDGE), lambda i: (i, 0))]
    + _w_specs([(D_EDGE, H), (1, H), (H, H), (1, H), (H, H), (1, H), (1, H),
                (1, H)]),
    out_specs=pl.BlockSpec((TRE, H), lambda i: (i, 0)),
    out_shape=jax.ShapeDtypeStruct((E, H), jnp.float32),
)


def _enc_node_body(x_ref, w1, b1, w2, b2, w3, b3, g, beta, wa, wb,
                   h_ref, ha_ref, hb_ref):
    l1 = jax.nn.relu(_dot(x_ref[...], w1[...]) + b1[...])
    l2 = jax.nn.relu(_dot(l1, w2[...]) + b2[...])
    l3 = _dot(l2, w3[...]) + b3[...]
    h = _ln(l3, g[...], beta[...])
    h_ref[...] = h
    ha_ref[...] = _dot(h, wa[...])
    hb_ref[...] = _dot(h, wb[...])


_enc_node = pl.pallas_call(
    _enc_node_body,
    grid=(NPAD // TRN,),
    in_specs=[pl.BlockSpec((TRN, H), lambda i: (i, 0))]
    + _w_specs([(H, H), (1, H), (H, H), (1, H), (H, H), (1, H), (1, H), (1, H),
                (H, H), (H, H)]),
    out_specs=[pl.BlockSpec((TRN, H), lambda i: (i, 0))] * 3,
    out_shape=[jax.ShapeDtypeStruct((NPAD, H), jnp.float32)] * 3,
)


def _edge_mlp_body(ga_ref, gb_ref, e_ref, w1c, b1, w2, b2, w3, b3, g, beta, wm,
                   e_out, m_out):
    e_in = e_ref[...]
    l1 = jax.nn.relu(ga_ref[...] + gb_ref[...] + _dot(e_in, w1c[...]) + b1[...])
    l2 = jax.nn.relu(_dot(l1, w2[...]) + b2[...])
    l3 = _dot(l2, w3[...]) + b3[...]
    e_new = _ln(l3, g[...], beta[...]) + e_in
    e_out[...] = e_new
    mm = _dot(e_new, wm[...])
    m_out[0] = mm[:, :HH]
    m_out[1] = mm[:, HH:]


_edge_mlp = pl.pallas_call(
    _edge_mlp_body,
    grid=(E // TRE,),
    in_specs=[pl.BlockSpec((TRE, H), lambda i: (i, 0))] * 3
    + _w_specs([(H, H), (1, H), (H, H), (1, H), (H, H), (1, H), (1, H), (1, H),
                (H, H)]),
    out_specs=[pl.BlockSpec((TRE, H), lambda i: (i, 0)),
               pl.BlockSpec((2, TRE, HH), lambda i: (0, i, 0))],
    out_shape=[jax.ShapeDtypeStruct((E, H), jnp.float32),
               jax.ShapeDtypeStruct((2, E, HH), jnp.float32)],
)


_AGG_SPECS = [pl.BlockSpec((1, TRN, HH), lambda i: (0, i, 0)),
              pl.BlockSpec((1, TRN, HH), lambda i: (1, i, 0))]


def _node_mlp_body(h_ref, a0_ref, a1_ref, w1, b1, w2, b2, w3, b3, g, beta,
                   wa, wb, h_out, ha_out, hb_out):
    h_in = h_ref[...]
    agg = jnp.concatenate([a0_ref[0], a1_ref[0]], axis=-1)
    l1 = jax.nn.relu(_dot(h_in, w1[...]) + agg + b1[...])
    l2 = jax.nn.relu(_dot(l1, w2[...]) + b2[...])
    l3 = _dot(l2, w3[...]) + b3[...]
    h_new = _ln(l3, g[...], beta[...]) + h_in
    h_out[...] = h_new
    ha_out[...] = _dot(h_new, wa[...])
    hb_out[...] = _dot(h_new, wb[...])


_node_mlp = pl.pallas_call(
    _node_mlp_body,
    grid=(NPAD // TRN,),
    in_specs=[pl.BlockSpec((TRN, H), lambda i: (i, 0))] + _AGG_SPECS
    + _w_specs([(H, H), (1, H), (H, H), (1, H), (H, H), (1, H), (1, H), (1, H),
                (H, H), (H, H)]),
    out_specs=[pl.BlockSpec((TRN, H), lambda i: (i, 0))] * 3,
    out_shape=[jax.ShapeDtypeStruct((NPAD, H), jnp.float32)] * 3,
)


def _node_last_body(h_ref, a0_ref, a1_ref, w1, b1, w2, b2, w3, b3, g, beta,
                    h_out):
    h_in = h_ref[...]
    agg = jnp.concatenate([a0_ref[0], a1_ref[0]], axis=-1)
    l1 = jax.nn.relu(_dot(h_in, w1[...]) + agg + b1[...])
    l2 = jax.nn.relu(_dot(l1, w2[...]) + b2[...])
    l3 = _dot(l2, w3[...]) + b3[...]
    h_out[...] = _ln(l3, g[...], beta[...]) + h_in


_node_last = pl.pallas_call(
    _node_last_body,
    grid=(NPAD // TRN,),
    in_specs=[pl.BlockSpec((TRN, H), lambda i: (i, 0))] + _AGG_SPECS
    + _w_specs([(H, H), (1, H), (H, H), (1, H), (H, H), (1, H), (1, H),
                (1, H)]),
    out_specs=pl.BlockSpec((TRN, H), lambda i: (i, 0)),
    out_shape=jax.ShapeDtypeStruct((NPAD, H), jnp.float32),
)


def _dec_body(h_ref, x_ref, w1, b1, w2, b2, w3, b3, o_ref):
    l1 = jax.nn.relu(_dot(h_ref[...], w1[...]) + b1[...])
    l2 = jax.nn.relu(_dot(l1, w2[...]) + b2[...])
    l3 = _dot(l2, w3[...]) + b3[...]
    o_ref[...] = l3 * 0.005 + x_ref[:, :3]


_dec = pl.pallas_call(
    _dec_body,
    grid=(NPAD // TRN,),
    in_specs=[pl.BlockSpec((TRN, H), lambda i: (i, 0)),
              pl.BlockSpec((TRN, H), lambda i: (i, 0))]
    + _w_specs([(H, H), (1, H), (H, H), (1, H), (H, 3), (1, 3)]),
    out_specs=pl.BlockSpec((TRN, 3), lambda i: (i, 0)),
    out_shape=jax.ShapeDtypeStruct((NPAD, 3), jnp.float32),
)


_sc_mesh = plsc.VectorSubcoreMesh(core_axis_name="c", subcore_axis_name="s")


@functools.partial(
    pl.kernel,
    out_type=[jax.ShapeDtypeStruct((E, H), jnp.float32),
              jax.ShapeDtypeStruct((E, H), jnp.float32)],
    mesh=_sc_mesh,
    scratch_types=[pltpu.VMEM((NCH, CH), jnp.int32),
                   pltpu.VMEM((NCH, CH), jnp.int32),
                   pltpu.VMEM((NB, CH, H), jnp.float32),
                   pltpu.VMEM((NB, CH, H), jnp.float32),
                   pltpu.SemaphoreType.DMA((NB,)),
                   pltpu.SemaphoreType.DMA((NB,)),
                   pltpu.SemaphoreType.DMA((NB,)),
                   pltpu.SemaphoreType.DMA((NB,))],
)
def _sc_gather(ha_hbm, hb_hbm, row3_hbm, col3_hbm, ga_hbm, gb_hbm,
               idx_a, idx_b, buf_a, buf_b, gsa, gsb, wsa, wsb):
    cid = lax.axis_index("c")
    sid = lax.axis_index("s")
    wid = sid * 2 + cid
    base = wid * PW
    pltpu.sync_copy(row3_hbm.at[wid], idx_a)
    pltpu.sync_copy(col3_hbm.at[wid], idx_b)
    for b in range(NB):
        pltpu.async_copy(ha_hbm.at[idx_a.at[b]], buf_a.at[b], gsa.at[b])
        pltpu.async_copy(hb_hbm.at[idx_b.at[b]], buf_b.at[b], gsb.at[b])

    @pl.loop(0, NCH, step=NB)
    def _round(step):
        for b in range(NB):
            ci = step + b
            s = base + ci * CH
            pltpu.make_async_copy(ha_hbm.at[idx_a.at[ci]], buf_a.at[b],
                                  gsa.at[b]).wait()
            pltpu.make_async_copy(hb_hbm.at[idx_b.at[ci]], buf_b.at[b],
                                  gsb.at[b]).wait()
            pltpu.async_copy(buf_a.at[b], ga_hbm.at[pl.ds(s, CH)], wsa.at[b])
            pltpu.async_copy(buf_b.at[b], gb_hbm.at[pl.ds(s, CH)], wsb.at[b])
        for b in range(NB):
            cj = step + NB + b
            pltpu.make_async_copy(buf_a.at[b], ga_hbm.at[pl.ds(base, CH)],
                                  wsa.at[b]).wait()
            pltpu.make_async_copy(buf_b.at[b], gb_hbm.at[pl.ds(base, CH)],
                                  wsb.at[b]).wait()

            @pl.when(cj < NCH)
            def _():
                pltpu.async_copy(ha_hbm.at[idx_a.at[cj]], buf_a.at[b],
                                 gsa.at[b])
                pltpu.async_copy(hb_hbm.at[idx_b.at[cj]], buf_b.at[b],
                                 gsb.at[b])


@functools.partial(
    pl.kernel,
    out_type=jax.ShapeDtypeStruct((2, NPAD, HH), jnp.float32),
    mesh=_sc_mesh,
    scratch_types=[pltpu.VMEM_SHARED((NPAD, HH), jnp.float32),
                   pltpu.VMEM((NCHS, CHS), jnp.int32),
                   pltpu.VMEM((NB, CHS, HH), jnp.float32),
                   pltpu.SemaphoreType.DMA((NB,))],
)
def _sc_scatter(ms_hbm, cols_hbm, zeros_hbm, agg_hbm, shared, idx, buf, lsem):
    cid = lax.axis_index("c")
    sid = lax.axis_index("s")
    base = sid * PWS
    pltpu.sync_copy(zeros_hbm.at[pl.ds(sid * RPT, RPT)],
                    shared.at[pl.ds(sid * RPT, RPT)])
    pltpu.sync_copy(cols_hbm.at[sid], idx)
    plsc.subcore_barrier()
    for b in range(NB):
        pltpu.async_copy(ms_hbm.at[cid, pl.ds(base + b * CHS, CHS)],
                         buf.at[b], lsem.at[b])

    @pl.loop(0, NCHS, step=NB)
    def _round(step):
        for b in range(NB):
            ci = step + b
            cj = ci + NB
            pltpu.make_async_copy(ms_hbm.at[cid, pl.ds(base, CHS)], buf.at[b],
                                  lsem.at[b]).wait()
            pltpu.sync_copy(buf.at[b], shared.at[idx.at[ci]], add=True)

            @pl.when(cj < NCHS)
            def _():
                pltpu.async_copy(ms_hbm.at[cid, pl.ds(base + cj * CHS, CHS)],
                                 buf.at[b], lsem.at[b])

    plsc.subcore_barrier()
    pltpu.sync_copy(shared.at[pl.ds(sid * RPT, RPT)],
                    agg_hbm.at[cid, pl.ds(sid * RPT, RPT)])


def _r(v):
    return v.reshape(1, -1)


def kernel(x, edge_index, edge_attr, params):
    row3 = edge_index[0].reshape(NW, NCH, CH)
    col3 = edge_index[1].reshape(NW, NCH, CH)
    cols = edge_index[1].reshape(16, NCHS, CHS)
    xp = jnp.pad(x, ((0, NPAD - N), (0, 0)))
    zeros_n = jnp.zeros((NPAD, HH), jnp.float32)

    sums = _bn_stats(edge_attr)
    mu = sums[0] / E
    var = sums[1] / E - mu * mu
    s = params["bn"]["gamma"] * lax.rsqrt(var + 1e-5)
    ep = params["edge_enc"]
    w1p = ep["W"][0] * s[:, None]
    b1p = ep["b"][0] + (params["bn"]["beta"] - mu * s) @ ep["W"][0]
    e = _enc_edge(edge_attr, w1p, _r(b1p), ep["W"][1], _r(ep["b"][1]),
                  ep["W"][2], _r(ep["b"][2]), _r(ep["g"]), _r(ep["beta"]))

    np_ = params["node_enc"]
    we0 = params["blocks"][0]["edge"]["W"][0]
    h, ha, hb = _enc_node(xp, np_["W"][0], _r(np_["b"][0]),
                          np_["W"][1], _r(np_["b"][1]),
                          np_["W"][2], _r(np_["b"][2]),
                          _r(np_["g"]), _r(np_["beta"]),
                          we0[:H], we0[H:2 * H])

    for i in range(MP):
        blk = params["blocks"][i]
        pe, pn = blk["edge"], blk["node"]
        ga, gb = _sc_gather(ha, hb, row3, col3)
        e, ms = _edge_mlp(ga, gb, e, pe["W"][0][2 * H:], _r(pe["b"][0]),
                          pe["W"][1], _r(pe["b"][1]), pe["W"][2],
                          _r(pe["b"][2]), _r(pe["g"]), _r(pe["beta"]),
                          pn["W"][0][H:])
        agg2 = _sc_scatter(ms, cols, zeros_n)
        nargs = (h, agg2, agg2, pn["W"][0][:H], _r(pn["b"][0]),
                 pn["W"][1], _r(pn["b"][1]), pn["W"][2], _r(pn["b"][2]),
                 _r(pn["g"]), _r(pn["beta"]))
        if i < MP - 1:
            wen = params["blocks"][i + 1]["edge"]["W"][0]
            h, ha, hb = _node_mlp(*nargs, wen[:H], wen[H:2 * H])
        else:
            h = _node_last(*nargs)

    d = params["dec"]
    out = _dec(h, xp, d["W"][0], _r(d["b"][0]), d["W"][1], _r(d["b"][1]),
               d["W"][2], _r(d["b"][2]))
    return out[:N]
